# Initial kernel scaffold; baseline (speedup 1.0000x reference)
import jax
import jax.numpy as jnp
from jax.experimental import pallas as pl


def kernel(user_id, timestamp, user_table, ts_table, buckets, norm_mean, norm_var):
    raise NotImplementedError("write your pallas kernel here")


# speedup vs baseline: 2.0239x; 2.0239x over previous
import functools

import jax
import jax.numpy as jnp
from jax import lax
from jax.experimental import pallas as pl
from jax.experimental.pallas import tpu as pltpu
from jax.experimental.pallas import tpu_sc as plsc

B = 16384
DIM = 32
NBUCKETS = 1000
OUT_COLS = 2 * DIM + 1

NC = 2
NS = 16
L = 16
NW = NC * NS
BPW = B // NW
NVEC = BPW // L
BKT_PAD = 1024
CONST_PAD = 128


def _sc_body(uid_hbm, ts_hbm, utab_hbm, ttab_hbm, bkt_hbm, consts_hbm,
             out_hbm, uid_v, ts_v, idx_v, bkt_v, consts_v, urows_v, trows_v,
             nrm_v, sem_u, sem_t):
    wid = lax.axis_index("s") * NC + lax.axis_index("c")
    base = wid * BPW

    pltpu.sync_copy(uid_hbm.at[pl.ds(base, BPW)], uid_v)
    user_gather = pltpu.async_copy(utab_hbm.at[uid_v], urows_v, sem_u)

    pltpu.sync_copy(ts_hbm.at[pl.ds(base, BPW)], ts_v)
    pltpu.sync_copy(bkt_hbm, bkt_v)
    pltpu.sync_copy(consts_hbm, consts_v)

    inv_step = consts_v[pl.ds(0, L)]
    mean = consts_v[pl.ds(L, L)]
    denom = consts_v[pl.ds(2 * L, L)]
    lanes = lax.iota(jnp.int32, L)
    zeros = jnp.zeros((L,), jnp.int32)

    for i in range(NVEC):
        t = ts_v[pl.ds(i * L, L)]
        g = jnp.clip((t * inv_step).astype(jnp.int32), 0, NBUCKETS - 2)
        blo = plsc.load_gather(bkt_v, [g])
        bhi = plsc.load_gather(bkt_v, [g + 1])
        idx = jnp.where(t < blo, g, jnp.where(t >= bhi, g + 2, g + 1))
        idx_v[pl.ds(i * L, L)] = idx
        plsc.store_scatter(nrm_v, [i * L + lanes, zeros], (t - mean) / denom)

    ts_gather = pltpu.async_copy(ttab_hbm.at[idx_v], trows_v, sem_t)

    user_gather.wait()
    pltpu.sync_copy(urows_v, out_hbm.at[pl.ds(base, BPW), pl.ds(0, DIM)])
    ts_gather.wait()
    pltpu.sync_copy(trows_v, out_hbm.at[pl.ds(base, BPW), pl.ds(DIM, DIM)])
    pltpu.sync_copy(nrm_v, out_hbm.at[pl.ds(base, BPW), pl.ds(2 * DIM, 1)])


@jax.jit
def _run(user_id, timestamp, user_table, ts_table, buckets_pad, consts):
    mesh = plsc.VectorSubcoreMesh(core_axis_name="c", subcore_axis_name="s")
    f = functools.partial(
        pl.kernel,
        mesh=mesh,
        compiler_params=pltpu.CompilerParams(
            needs_layout_passes=False, use_tc_tiling_on_sc=False),
        out_type=jax.ShapeDtypeStruct((B, OUT_COLS), jnp.float32),
        scratch_types=[
            pltpu.VMEM((BPW,), jnp.int32),
            pltpu.VMEM((BPW,), jnp.float32),
            pltpu.VMEM((BPW,), jnp.int32),
            pltpu.VMEM((BKT_PAD,), jnp.float32),
            pltpu.VMEM((CONST_PAD,), jnp.float32),
            pltpu.VMEM((BPW, DIM), jnp.float32),
            pltpu.VMEM((BPW, DIM), jnp.float32),
            pltpu.VMEM((BPW, 1), jnp.float32),
            pltpu.SemaphoreType.DMA,
            pltpu.SemaphoreType.DMA,
        ],
    )(_sc_body)
    return f(user_id, timestamp, user_table, ts_table, buckets_pad, consts)


def kernel(user_id, timestamp, user_table, ts_table, buckets, norm_mean,
           norm_var):
    n = buckets.shape[0]
    inv_step = (n - 1.0) / (buckets[-1] - buckets[0])
    denom = jnp.sqrt(norm_var + 1e-6)
    consts = jnp.concatenate([
        jnp.full((L,), inv_step, jnp.float32),
        jnp.full((L,), norm_mean, jnp.float32),
        jnp.full((L,), denom, jnp.float32),
        jnp.zeros((CONST_PAD - 3 * L,), jnp.float32),
    ])
    buckets_pad = jnp.concatenate(
        [buckets, jnp.full((BKT_PAD - n,), jnp.inf, jnp.float32)])
    return _run(user_id, timestamp, user_table, ts_table, buckets_pad,
                consts)
